# Initial kernel scaffold; baseline (speedup 1.0000x reference)
import jax
import jax.numpy as jnp
from jax.experimental import pallas as pl


def kernel(h, W1, b1, W2, b2, edge_index, comment_assignment):
    raise NotImplementedError("write your pallas kernel here")


# speedup vs baseline: 4.8387x; 4.8387x over previous
import functools

import jax
import jax.numpy as jnp
from jax import lax
from jax.experimental import pallas as pl
from jax.experimental.pallas import tpu as pltpu
from jax.experimental.pallas import tpu_sc as plsc

_EPS = 1e-15
_N = 10000
_E = 320000
_D = 128
_B = 8
_M = 20
_NSEG = _B * _M

_BS = 1000
_K = 80
_NW = 32
_EW = _E // _NW
_NCH = _EW // _K
_NP = 10240
_RPS = _NP // 16


def _nrm(x):
    return jnp.sqrt(jnp.sum(x * x, axis=-1, keepdims=True) + _EPS)


def _artanh(x):
    x = jnp.clip(x, -1.0 + 1e-7, 1.0 - 1e-7)
    return 0.5 * jnp.log((1.0 + x) / (1.0 - x))


def _expmap0(u):
    n = _nrm(u)
    return jnp.tanh(n) * u / n


def _logmap0(p):
    n = _nrm(p)
    return _artanh(n) * p / n


def _proj(x):
    maxnorm = 1.0 - 1e-5
    n = _nrm(x)
    return jnp.where(n > maxnorm, x / n * maxnorm, x)


def _mobius_add(x, y):
    x2 = jnp.sum(x * x, -1, keepdims=True)
    y2 = jnp.sum(y * y, -1, keepdims=True)
    xy = jnp.sum(x * y, -1, keepdims=True)
    num = (1.0 + 2.0 * xy + y2) * x + (1.0 - x2) * y
    den = 1.0 + 2.0 * xy + x2 * y2
    return num / jnp.maximum(den, _EPS)


def _hyplinear_log(x, W, b):
    xn = _nrm(x)
    mx = lax.dot_general(x, W, (((1,), (1,)), ((), ())),
                         preferred_element_type=jnp.float32)
    mxn = _nrm(mx)
    mv = _proj(jnp.tanh(mxn / xn * _artanh(xn)) * mx / mxn)
    hb = _proj(_expmap0(b))
    res = _proj(_mobius_add(mv, hb))
    return _logmap0(res)


def _post_agg(acc, deg):
    s = acc[0] + acc[1]
    d = deg[0, :, 0:1] + deg[1, :, 0:1]
    agg = s / jnp.maximum(d, 1.0)
    out = _proj(_expmap0(agg))
    xt = jax.nn.relu(_logmap0(out))
    return _proj(_expmap0(xt))


def _stage1_body(h_ref, W_ref, b_ref, o_ref):
    x = _proj(_expmap0(h_ref[...]))
    o_ref[...] = _hyplinear_log(x, W_ref[...], b_ref[...])


def _stage2_body(acc_ref, deg_ref, W_ref, b_ref, o_ref):
    x = _post_agg(acc_ref[...], deg_ref[...])
    o_ref[...] = _hyplinear_log(x, W_ref[...], b_ref[...])


def _stage3_body(acc_ref, deg_ref, ca_ref, o_ref, cnt_ref):
    i = pl.program_id(0)
    x = _post_agg(acc_ref[...], deg_ref[...])
    ht = _logmap0(_proj(x))
    ca = ca_ref[...]
    onehot = (ca == lax.broadcasted_iota(jnp.int32, (_BS, _NSEG), 1)
              ).astype(jnp.float32)

    @pl.when(i == 0)
    def _():
        o_ref[...] = jnp.zeros_like(o_ref)
        cnt_ref[...] = jnp.zeros_like(cnt_ref)

    o_ref[...] += lax.dot_general(onehot, ht, (((0,), (0,)), ((), ())),
                                  preferred_element_type=jnp.float32)
    cnt_ref[...] += lax.dot_general(onehot, jnp.ones((_BS, _D), jnp.float32),
                                    (((0,), (0,)), ((), ())),
                                    preferred_element_type=jnp.float32)

    @pl.when(i == pl.num_programs(0) - 1)
    def _():
        mean = o_ref[...] / jnp.maximum(cnt_ref[...], 1.0)
        o_ref[...] = _proj(_expmap0(mean))


_stage1 = pl.pallas_call(
    _stage1_body,
    grid=(_N // _BS,),
    in_specs=[pl.BlockSpec((_BS, _D), lambda i: (i, 0)),
              pl.BlockSpec((_D, _D), lambda i: (0, 0)),
              pl.BlockSpec((1, _D), lambda i: (0, 0))],
    out_specs=pl.BlockSpec((_BS, _D), lambda i: (i, 0)),
    out_shape=jax.ShapeDtypeStruct((_N, _D), jnp.float32),
)

_stage2 = pl.pallas_call(
    _stage2_body,
    grid=(_N // _BS,),
    in_specs=[pl.BlockSpec((2, _BS, _D), lambda i: (0, i, 0)),
              pl.BlockSpec((2, _BS, _D), lambda i: (0, i, 0)),
              pl.BlockSpec((_D, _D), lambda i: (0, 0)),
              pl.BlockSpec((1, _D), lambda i: (0, 0))],
    out_specs=pl.BlockSpec((_BS, _D), lambda i: (i, 0)),
    out_shape=jax.ShapeDtypeStruct((_N, _D), jnp.float32),
)

_stage3 = pl.pallas_call(
    _stage3_body,
    grid=(_N // _BS,),
    in_specs=[pl.BlockSpec((2, _BS, _D), lambda i: (0, i, 0)),
              pl.BlockSpec((2, _BS, _D), lambda i: (0, i, 0)),
              pl.BlockSpec((_BS, 1), lambda i: (i, 0))],
    out_specs=pl.BlockSpec((_NSEG, _D), lambda i: (0, 0)),
    out_shape=jax.ShapeDtypeStruct((_NSEG, _D), jnp.float32),
    scratch_shapes=[pltpu.VMEM((_NSEG, _D), jnp.float32)],
)


def _sc_agg_body(xt_hbm, src_hbm, dst_hbm, znd_hbm,
                 acc_out, src_v, dst_v, rows_v, acc_sh, sem):
    cid = lax.axis_index("c")
    sid = lax.axis_index("s")
    row0 = sid * _RPS
    pltpu.sync_copy(znd_hbm.at[pl.ds(row0, _RPS)], acc_sh.at[pl.ds(row0, _RPS)])
    plsc.subcore_barrier()

    base = (sid * 2 + cid) * _EW

    @pl.loop(0, _NCH)
    def _(ci):
        off = base + ci * _K
        pltpu.sync_copy(src_hbm.at[pl.ds(off, _K)], src_v)
        pltpu.sync_copy(dst_hbm.at[pl.ds(off, _K)], dst_v)
        pltpu.async_copy(xt_hbm.at[src_v], rows_v, sem).wait()
        pltpu.sync_copy(rows_v, acc_sh.at[dst_v], add=True)

    plsc.subcore_barrier()
    pltpu.sync_copy(acc_sh.at[pl.ds(row0, _RPS)],
                    acc_out.at[cid, pl.ds(row0, _RPS)])


def _sc_deg_body(dst_hbm, znd_hbm, ones_hbm,
                 deg_out, dst_v, ones_v, deg_sh, sem):
    cid = lax.axis_index("c")
    sid = lax.axis_index("s")
    row0 = sid * _RPS
    pltpu.sync_copy(znd_hbm.at[pl.ds(row0, _RPS)], deg_sh.at[pl.ds(row0, _RPS)])
    pltpu.sync_copy(ones_hbm, ones_v)
    plsc.subcore_barrier()

    base = (sid * 2 + cid) * _EW

    @pl.loop(0, _NCH)
    def _(ci):
        off = base + ci * _K
        pltpu.sync_copy(dst_hbm.at[pl.ds(off, _K)], dst_v)
        pltpu.sync_copy(ones_v, deg_sh.at[dst_v], add=True)

    plsc.subcore_barrier()
    pltpu.sync_copy(deg_sh.at[pl.ds(row0, _RPS)],
                    deg_out.at[cid, pl.ds(row0, _RPS)])


@functools.cache
def _get_sc_kernels():
    mesh = plsc.VectorSubcoreMesh(core_axis_name="c", subcore_axis_name="s")
    agg = pl.kernel(
        _sc_agg_body,
        out_type=jax.ShapeDtypeStruct((2, _NP, _D), jnp.float32),
        mesh=mesh,
        scratch_types=[
            pltpu.VMEM((_K,), jnp.int32),
            pltpu.VMEM((_K,), jnp.int32),
            pltpu.VMEM((_K, _D), jnp.float32),
            pltpu.VMEM_SHARED((_NP, _D), jnp.float32),
            pltpu.SemaphoreType.DMA,
        ],
    )
    deg = pl.kernel(
        _sc_deg_body,
        out_type=jax.ShapeDtypeStruct((2, _NP, _D), jnp.float32),
        mesh=mesh,
        scratch_types=[
            pltpu.VMEM((_K,), jnp.int32),
            pltpu.VMEM((_K, _D), jnp.float32),
            pltpu.VMEM_SHARED((_NP, _D), jnp.float32),
            pltpu.SemaphoreType.DMA,
        ],
    )
    return agg, deg


def kernel(h, W1, b1, W2, b2, edge_index, comment_assignment):
    src = edge_index[0].astype(jnp.int32)
    dst = edge_index[1].astype(jnp.int32)
    ca = comment_assignment.astype(jnp.int32).reshape(_N, 1)
    zeros_nd = jnp.zeros((_NP, _D), jnp.float32)
    ones_k = jnp.ones((_K, _D), jnp.float32)

    sc_agg, sc_deg = _get_sc_kernels()
    deg = sc_deg(dst, zeros_nd, ones_k)
    xt1 = _stage1(h, W1, b1.reshape(1, _D))
    acc1 = sc_agg(xt1, src, dst, zeros_nd)
    xt2 = _stage2(acc1, deg, W2, b2.reshape(1, _D))
    acc2 = sc_agg(xt2, src, dst, zeros_nd)
    out = _stage3(acc2, deg, ca)
    return out.reshape(_B, _M, _D)


# speedup vs baseline: 9.5605x; 1.9758x over previous
import functools

import jax
import jax.numpy as jnp
from jax import lax
from jax.experimental import pallas as pl
from jax.experimental.pallas import tpu as pltpu
from jax.experimental.pallas import tpu_sc as plsc

_EPS = 1e-15
_N = 10000
_E = 320000
_D = 128
_B = 8
_M = 20
_NSEG = _B * _M

_BS = 1000
_K = 80
_NW = 32
_EW = _E // _NW
_NCH = _EW // _K
_NP = 10240
_RPS = _NP // 16


def _nrm(x):
    return jnp.sqrt(jnp.sum(x * x, axis=-1, keepdims=True) + _EPS)


def _artanh(x):
    x = jnp.clip(x, -1.0 + 1e-7, 1.0 - 1e-7)
    return 0.5 * jnp.log((1.0 + x) / (1.0 - x))


def _expmap0(u):
    n = _nrm(u)
    return jnp.tanh(n) * u / n


def _logmap0(p):
    n = _nrm(p)
    return _artanh(n) * p / n


def _proj(x):
    maxnorm = 1.0 - 1e-5
    n = _nrm(x)
    return jnp.where(n > maxnorm, x / n * maxnorm, x)


def _mobius_add(x, y):
    x2 = jnp.sum(x * x, -1, keepdims=True)
    y2 = jnp.sum(y * y, -1, keepdims=True)
    xy = jnp.sum(x * y, -1, keepdims=True)
    num = (1.0 + 2.0 * xy + y2) * x + (1.0 - x2) * y
    den = 1.0 + 2.0 * xy + x2 * y2
    return num / jnp.maximum(den, _EPS)


def _hyplinear_log(x, W, b):
    xn = _nrm(x)
    mx = lax.dot_general(x, W, (((1,), (1,)), ((), ())),
                         preferred_element_type=jnp.float32)
    mxn = _nrm(mx)
    mv = _proj(jnp.tanh(mxn / xn * _artanh(xn)) * mx / mxn)
    hb = _proj(_expmap0(b))
    res = _proj(_mobius_add(mv, hb))
    return _logmap0(res)


def _post_agg(acc, deg):
    s = acc[0] + acc[1]
    d = deg[0, :, 0:1] + deg[1, :, 0:1]
    agg = s / jnp.maximum(d, 1.0)
    out = _proj(_expmap0(agg))
    xt = jax.nn.relu(_logmap0(out))
    return _proj(_expmap0(xt))


def _stage1_body(h_ref, W_ref, b_ref, o_ref):
    x = _proj(_expmap0(h_ref[...]))
    o_ref[...] = _hyplinear_log(x, W_ref[...], b_ref[...])


def _stage2_body(acc_ref, deg_ref, W_ref, b_ref, o_ref):
    x = _post_agg(acc_ref[...], deg_ref[...])
    o_ref[...] = _hyplinear_log(x, W_ref[...], b_ref[...])


def _stage3_body(acc_ref, deg_ref, ca_ref, o_ref, cnt_ref):
    i = pl.program_id(0)
    x = _post_agg(acc_ref[...], deg_ref[...])
    ht = _logmap0(_proj(x))
    ca = ca_ref[...]
    onehot = (ca == lax.broadcasted_iota(jnp.int32, (_BS, _NSEG), 1)
              ).astype(jnp.float32)

    @pl.when(i == 0)
    def _():
        o_ref[...] = jnp.zeros_like(o_ref)
        cnt_ref[...] = jnp.zeros_like(cnt_ref)

    o_ref[...] += lax.dot_general(onehot, ht, (((0,), (0,)), ((), ())),
                                  preferred_element_type=jnp.float32)
    cnt_ref[...] += lax.dot_general(onehot, jnp.ones((_BS, _D), jnp.float32),
                                    (((0,), (0,)), ((), ())),
                                    preferred_element_type=jnp.float32)

    @pl.when(i == pl.num_programs(0) - 1)
    def _():
        mean = o_ref[...] / jnp.maximum(cnt_ref[...], 1.0)
        o_ref[...] = _proj(_expmap0(mean))


_stage1 = pl.pallas_call(
    _stage1_body,
    grid=(_N // _BS,),
    in_specs=[pl.BlockSpec((_BS, _D), lambda i: (i, 0)),
              pl.BlockSpec((_D, _D), lambda i: (0, 0)),
              pl.BlockSpec((1, _D), lambda i: (0, 0))],
    out_specs=pl.BlockSpec((_BS, _D), lambda i: (i, 0)),
    out_shape=jax.ShapeDtypeStruct((_N, _D), jnp.float32),
)

_stage2 = pl.pallas_call(
    _stage2_body,
    grid=(_N // _BS,),
    in_specs=[pl.BlockSpec((2, _BS, _D), lambda i: (0, i, 0)),
              pl.BlockSpec((2, _BS, _D), lambda i: (0, i, 0)),
              pl.BlockSpec((_D, _D), lambda i: (0, 0)),
              pl.BlockSpec((1, _D), lambda i: (0, 0))],
    out_specs=pl.BlockSpec((_BS, _D), lambda i: (i, 0)),
    out_shape=jax.ShapeDtypeStruct((_N, _D), jnp.float32),
)

_stage3 = pl.pallas_call(
    _stage3_body,
    grid=(_N // _BS,),
    in_specs=[pl.BlockSpec((2, _BS, _D), lambda i: (0, i, 0)),
              pl.BlockSpec((2, _BS, _D), lambda i: (0, i, 0)),
              pl.BlockSpec((_BS, 1), lambda i: (i, 0))],
    out_specs=pl.BlockSpec((_NSEG, _D), lambda i: (0, 0)),
    out_shape=jax.ShapeDtypeStruct((_NSEG, _D), jnp.float32),
    scratch_shapes=[pltpu.VMEM((_NSEG, _D), jnp.float32)],
)


def _sc_agg_body(xt_hbm, src_hbm, dst_hbm, znd_hbm,
                 acc_out, src_v, dst_v, rows0_v, rows1_v, acc_sh,
                 sem0, sem1):
    cid = lax.axis_index("c")
    sid = lax.axis_index("s")
    row0 = sid * _RPS
    wid = sid * 2 + cid
    pltpu.sync_copy(znd_hbm.at[pl.ds(row0, _RPS)], acc_sh.at[pl.ds(row0, _RPS)])
    pltpu.sync_copy(src_hbm.at[pl.ds(wid * _EW, _EW)], src_v)
    pltpu.sync_copy(dst_hbm.at[wid], dst_v)
    plsc.subcore_barrier()

    def _gather(j, buf, sem):
        return pltpu.make_async_copy(
            xt_hbm.at[src_v.at[pl.ds(j * _K, _K)]], buf, sem)

    bufs = ((rows0_v, sem0), (rows1_v, sem1))
    for b, (buf, sem) in enumerate(bufs):
        _gather(b, buf, sem).start()

    @pl.loop(0, _NCH - 2 - (_NCH % 2), step=2)
    def _(ci):
        for b, (buf, sem) in enumerate(bufs):
            j = ci + b
            _gather(j, buf, sem).wait()
            pltpu.sync_copy(buf, acc_sh.at[dst_v.at[j]], add=True)
            _gather(j + 2, buf, sem).start()

    for b, (buf, sem) in enumerate(bufs):
        j = _NCH - 2 - (_NCH % 2) + b
        _gather(j, buf, sem).wait()
        pltpu.sync_copy(buf, acc_sh.at[dst_v.at[j]], add=True)
    if _NCH % 2:
        j = _NCH - 1
        g = _gather(j, rows0_v, sem0)
        g.start()
        g.wait()
        pltpu.sync_copy(rows0_v, acc_sh.at[dst_v.at[j]], add=True)

    plsc.subcore_barrier()
    pltpu.sync_copy(acc_sh.at[pl.ds(row0, _RPS)],
                    acc_out.at[cid, pl.ds(row0, _RPS)])


def _sc_deg_body(dst_hbm, znd_hbm, ones_hbm,
                 deg_out, dst_v, ones_v, deg_sh, sem):
    cid = lax.axis_index("c")
    sid = lax.axis_index("s")
    row0 = sid * _RPS
    wid = sid * 2 + cid
    pltpu.sync_copy(znd_hbm.at[pl.ds(row0, _RPS)], deg_sh.at[pl.ds(row0, _RPS)])
    pltpu.sync_copy(ones_hbm, ones_v)
    pltpu.sync_copy(dst_hbm.at[wid], dst_v)
    plsc.subcore_barrier()

    @pl.loop(0, _NCH)
    def _(ci):
        pltpu.sync_copy(ones_v, deg_sh.at[dst_v.at[ci]], add=True)

    plsc.subcore_barrier()
    pltpu.sync_copy(deg_sh.at[pl.ds(row0, _RPS)],
                    deg_out.at[cid, pl.ds(row0, _RPS)])


@functools.cache
def _get_sc_kernels():
    mesh = plsc.VectorSubcoreMesh(core_axis_name="c", subcore_axis_name="s")
    agg = pl.kernel(
        _sc_agg_body,
        out_type=jax.ShapeDtypeStruct((2, _NP, _D), jnp.float32),
        mesh=mesh,
        scratch_types=[
            pltpu.VMEM((_EW,), jnp.int32),
            pltpu.VMEM((_NCH, _K), jnp.int32),
            pltpu.VMEM((_K, _D), jnp.float32),
            pltpu.VMEM((_K, _D), jnp.float32),
            pltpu.VMEM_SHARED((_NP, _D), jnp.float32),
            pltpu.SemaphoreType.DMA,
            pltpu.SemaphoreType.DMA,
        ],
    )
    deg = pl.kernel(
        _sc_deg_body,
        out_type=jax.ShapeDtypeStruct((2, _NP, _D), jnp.float32),
        mesh=mesh,
        scratch_types=[
            pltpu.VMEM((_NCH, _K), jnp.int32),
            pltpu.VMEM((_K, _D), jnp.float32),
            pltpu.VMEM_SHARED((_NP, _D), jnp.float32),
            pltpu.SemaphoreType.DMA,
        ],
    )
    return agg, deg


def kernel(h, W1, b1, W2, b2, edge_index, comment_assignment):
    src = edge_index[0].astype(jnp.int32)
    dst = edge_index[1].astype(jnp.int32).reshape(_NW, _NCH, _K)
    ca = comment_assignment.astype(jnp.int32).reshape(_N, 1)
    zeros_nd = jnp.zeros((_NP, _D), jnp.float32)
    ones_k = jnp.ones((_K, _D), jnp.float32)

    sc_agg, sc_deg = _get_sc_kernels()
    deg = sc_deg(dst, zeros_nd, ones_k)
    xt1 = _stage1(h, W1, b1.reshape(1, _D))
    acc1 = sc_agg(xt1, src, dst, zeros_nd)
    xt2 = _stage2(acc1, deg, W2, b2.reshape(1, _D))
    acc2 = sc_agg(xt2, src, dst, zeros_nd)
    out = _stage3(acc2, deg, ca)
    return out.reshape(_B, _M, _D)


# speedup vs baseline: 10.2754x; 1.0748x over previous
import functools

import jax
import jax.numpy as jnp
from jax import lax
from jax.experimental import pallas as pl
from jax.experimental.pallas import tpu as pltpu
from jax.experimental.pallas import tpu_sc as plsc

_EPS = 1e-15
_N = 10000
_E = 320000
_D = 128
_B = 8
_M = 20
_NSEG = _B * _M

_BS = 1000
_K = 40
_NW = 32
_EW = _E // _NW
_NCH = _EW // _K
_NP = 10240
_RPS = _NP // 16


def _nrm(x):
    return jnp.sqrt(jnp.sum(x * x, axis=-1, keepdims=True) + _EPS)


def _artanh(x):
    x = jnp.clip(x, -1.0 + 1e-7, 1.0 - 1e-7)
    return 0.5 * jnp.log((1.0 + x) / (1.0 - x))


def _expmap0(u):
    n = _nrm(u)
    return jnp.tanh(n) * u / n


def _logmap0(p):
    n = _nrm(p)
    return _artanh(n) * p / n


def _proj(x):
    maxnorm = 1.0 - 1e-5
    n = _nrm(x)
    return jnp.where(n > maxnorm, x / n * maxnorm, x)


def _mobius_add(x, y):
    x2 = jnp.sum(x * x, -1, keepdims=True)
    y2 = jnp.sum(y * y, -1, keepdims=True)
    xy = jnp.sum(x * y, -1, keepdims=True)
    num = (1.0 + 2.0 * xy + y2) * x + (1.0 - x2) * y
    den = 1.0 + 2.0 * xy + x2 * y2
    return num / jnp.maximum(den, _EPS)


def _hyplinear_log(x, W, b):
    xn = _nrm(x)
    mx = lax.dot_general(x, W, (((1,), (1,)), ((), ())),
                         preferred_element_type=jnp.float32)
    mxn = _nrm(mx)
    mv = _proj(jnp.tanh(mxn / xn * _artanh(xn)) * mx / mxn)
    hb = _proj(_expmap0(b))
    res = _proj(_mobius_add(mv, hb))
    return _logmap0(res)


def _post_agg(acc, deg):
    s = acc[0] + acc[1]
    d = deg[0, :, 0:1] + deg[1, :, 0:1]
    agg = s / jnp.maximum(d, 1.0)
    out = _proj(_expmap0(agg))
    xt = jax.nn.relu(_logmap0(out))
    return _proj(_expmap0(xt))


def _stage1_body(h_ref, W_ref, b_ref, o_ref):
    x = _proj(_expmap0(h_ref[...]))
    o_ref[...] = _hyplinear_log(x, W_ref[...], b_ref[...])


def _stage2_body(acc_ref, deg_ref, W_ref, b_ref, o_ref):
    x = _post_agg(acc_ref[...], deg_ref[...])
    o_ref[...] = _hyplinear_log(x, W_ref[...], b_ref[...])


def _stage3_body(acc_ref, deg_ref, ca_ref, o_ref, cnt_ref):
    i = pl.program_id(0)
    x = _post_agg(acc_ref[...], deg_ref[...])
    ht = _logmap0(_proj(x))
    ca = ca_ref[...]
    onehot = (ca == lax.broadcasted_iota(jnp.int32, (_BS, _NSEG), 1)
              ).astype(jnp.float32)

    @pl.when(i == 0)
    def _():
        o_ref[...] = jnp.zeros_like(o_ref)
        cnt_ref[...] = jnp.zeros_like(cnt_ref)

    o_ref[...] += lax.dot_general(onehot, ht, (((0,), (0,)), ((), ())),
                                  preferred_element_type=jnp.float32)
    cnt_ref[...] += lax.dot_general(onehot, jnp.ones((_BS, _D), jnp.float32),
                                    (((0,), (0,)), ((), ())),
                                    preferred_element_type=jnp.float32)

    @pl.when(i == pl.num_programs(0) - 1)
    def _():
        mean = o_ref[...] / jnp.maximum(cnt_ref[...], 1.0)
        o_ref[...] = _proj(_expmap0(mean))


_stage1 = pl.pallas_call(
    _stage1_body,
    grid=(_N // _BS,),
    in_specs=[pl.BlockSpec((_BS, _D), lambda i: (i, 0)),
              pl.BlockSpec((_D, _D), lambda i: (0, 0)),
              pl.BlockSpec((1, _D), lambda i: (0, 0))],
    out_specs=pl.BlockSpec((_BS, _D), lambda i: (i, 0)),
    out_shape=jax.ShapeDtypeStruct((_N, _D), jnp.float32),
)

_stage2 = pl.pallas_call(
    _stage2_body,
    grid=(_N // _BS,),
    in_specs=[pl.BlockSpec((2, _BS, _D), lambda i: (0, i, 0)),
              pl.BlockSpec((2, _BS, _D), lambda i: (0, i, 0)),
              pl.BlockSpec((_D, _D), lambda i: (0, 0)),
              pl.BlockSpec((1, _D), lambda i: (0, 0))],
    out_specs=pl.BlockSpec((_BS, _D), lambda i: (i, 0)),
    out_shape=jax.ShapeDtypeStruct((_N, _D), jnp.float32),
)

_stage3 = pl.pallas_call(
    _stage3_body,
    grid=(_N // _BS,),
    in_specs=[pl.BlockSpec((2, _BS, _D), lambda i: (0, i, 0)),
              pl.BlockSpec((2, _BS, _D), lambda i: (0, i, 0)),
              pl.BlockSpec((_BS, 1), lambda i: (i, 0))],
    out_specs=pl.BlockSpec((_NSEG, _D), lambda i: (0, 0)),
    out_shape=jax.ShapeDtypeStruct((_NSEG, _D), jnp.float32),
    scratch_shapes=[pltpu.VMEM((_NSEG, _D), jnp.float32)],
)


_NBUF = 5
_GLEAD = 3
_KD = 80
_NCHD = _EW // _KD
_DWIN = 8


def _sc_agg_body(xt_hbm, src_hbm, dst_hbm, znd_hbm, acc_out,
                 src_v, dst_v, r0, r1, r2, r3, r4, acc_sh,
                 g0, g1, g2, g3, g4, s0, s1, s2, s3, s4):
    cid = lax.axis_index("c")
    sid = lax.axis_index("s")
    row0 = sid * _RPS
    wid = sid * 2 + cid
    rows = (r0, r1, r2, r3, r4)
    gsem = (g0, g1, g2, g3, g4)
    ssem = (s0, s1, s2, s3, s4)

    pltpu.sync_copy(src_hbm.at[pl.ds(wid * _EW, _EW)], src_v)
    pltpu.sync_copy(dst_hbm.at[pl.ds(wid * _EW, _EW)], dst_v)

    def _gather(j, b):
        return pltpu.make_async_copy(
            xt_hbm.at[src_v.at[pl.ds(j * _K, _K)]], rows[b], gsem[b])

    def _scatter(j, b):
        return pltpu.make_async_copy(
            rows[b], acc_sh.at[dst_v.at[pl.ds(j * _K, _K)]], ssem[b])

    def _scatter_start(j, b):
        pltpu.async_copy(rows[b], acc_sh.at[dst_v.at[pl.ds(j * _K, _K)]],
                         ssem[b], add=True)

    for c in range(_GLEAD):
        _gather(c, c).start()
    pltpu.sync_copy(znd_hbm.at[pl.ds(row0, _RPS)], acc_sh.at[pl.ds(row0, _RPS)])
    plsc.subcore_barrier()

    for j in range(2):
        _gather(j, j % _NBUF).wait()
        _scatter_start(j, j % _NBUF)
        bn = (j + _GLEAD) % _NBUF
        _gather(j + _GLEAD, bn).start()

    @pl.loop(2, _NCH - _GLEAD, step=_NBUF)
    def _(ci):
        for u in range(_NBUF):
            b = (2 + u) % _NBUF
            bn = (2 + u + _GLEAD) % _NBUF
            j = ci + u
            _gather(j, b).wait()
            _scatter_start(j, b)
            _scatter(j - 2, bn).wait()
            _gather(j + _GLEAD, bn).start()

    for j in range(_NCH - _GLEAD, _NCH):
        b = j % _NBUF
        bn = (j + _GLEAD) % _NBUF
        _gather(j, b).wait()
        _scatter_start(j, b)
        _scatter(j - 2, bn).wait()
    for j in range(_NCH - 2, _NCH):
        _scatter(j, j % _NBUF).wait()

    plsc.subcore_barrier()
    pltpu.sync_copy(acc_sh.at[pl.ds(row0, _RPS)],
                    acc_out.at[cid, pl.ds(row0, _RPS)])


def _sc_deg_body(dst_hbm, znd_hbm, ones_hbm,
                 deg_out, dst_v, ones_v, deg_sh, sem):
    cid = lax.axis_index("c")
    sid = lax.axis_index("s")
    row0 = sid * _RPS
    wid = sid * 2 + cid
    pltpu.sync_copy(znd_hbm.at[pl.ds(row0, _RPS)], deg_sh.at[pl.ds(row0, _RPS)])
    pltpu.sync_copy(ones_hbm, ones_v)
    pltpu.sync_copy(dst_hbm.at[pl.ds(wid * _EW, _EW)], dst_v)
    plsc.subcore_barrier()

    def _sc(j):
        return pltpu.make_async_copy(
            ones_v, deg_sh.at[dst_v.at[pl.ds(j * _KD, _KD)]], sem)

    def _sc_start(j):
        pltpu.async_copy(ones_v, deg_sh.at[dst_v.at[pl.ds(j * _KD, _KD)]],
                         sem, add=True)

    for j in range(_DWIN):
        _sc_start(j)

    @pl.loop(_DWIN, _NCHD)
    def _(j):
        _sc(j - _DWIN).wait()
        _sc_start(j)

    for j in range(_NCHD - _DWIN, _NCHD):
        _sc(j).wait()

    plsc.subcore_barrier()
    pltpu.sync_copy(deg_sh.at[pl.ds(row0, _RPS)],
                    deg_out.at[cid, pl.ds(row0, _RPS)])


@functools.cache
def _get_sc_kernels():
    mesh = plsc.VectorSubcoreMesh(core_axis_name="c", subcore_axis_name="s")
    agg = pl.kernel(
        _sc_agg_body,
        out_type=jax.ShapeDtypeStruct((2, _NP, _D), jnp.float32),
        mesh=mesh,
        scratch_types=(
            [pltpu.VMEM((_EW,), jnp.int32),
             pltpu.VMEM((_EW,), jnp.int32)]
            + [pltpu.VMEM((_K, _D), jnp.float32) for _ in range(_NBUF)]
            + [pltpu.VMEM_SHARED((_NP, _D), jnp.float32)]
            + [pltpu.SemaphoreType.DMA] * (2 * _NBUF)
        ),
    )
    deg = pl.kernel(
        _sc_deg_body,
        out_type=jax.ShapeDtypeStruct((2, _NP, _D), jnp.float32),
        mesh=mesh,
        scratch_types=[
            pltpu.VMEM((_EW,), jnp.int32),
            pltpu.VMEM((_KD, _D), jnp.float32),
            pltpu.VMEM_SHARED((_NP, _D), jnp.float32),
            pltpu.SemaphoreType.DMA,
        ],
    )
    return agg, deg


def kernel(h, W1, b1, W2, b2, edge_index, comment_assignment):
    src = edge_index[0].astype(jnp.int32)
    dst = edge_index[1].astype(jnp.int32)
    ca = comment_assignment.astype(jnp.int32).reshape(_N, 1)
    zeros_nd = jnp.zeros((_NP, _D), jnp.float32)
    ones_k = jnp.ones((_KD, _D), jnp.float32)

    sc_agg, sc_deg = _get_sc_kernels()
    deg = sc_deg(dst, zeros_nd, ones_k)
    xt1 = _stage1(h, W1, b1.reshape(1, _D))
    acc1 = sc_agg(xt1, src, dst, zeros_nd)
    xt2 = _stage2(acc1, deg, W2, b2.reshape(1, _D))
    acc2 = sc_agg(xt2, src, dst, zeros_nd)
    out = _stage3(acc2, deg, ca)
    return out.reshape(_B, _M, _D)


# speedup vs baseline: 10.8261x; 1.0536x over previous
import functools

import jax
import jax.numpy as jnp
from jax import lax
from jax.experimental import pallas as pl
from jax.experimental.pallas import tpu as pltpu
from jax.experimental.pallas import tpu_sc as plsc

_EPS = 1e-15
_N = 10000
_E = 320000
_D = 128
_B = 8
_M = 20
_NSEG = _B * _M

_BS = 1000
_K = 40
_NW = 32
_EW = _E // _NW
_NCH = _EW // _K
_NP = 10240
_RPS = _NP // 16


def _nrm(x):
    return jnp.sqrt(jnp.sum(x * x, axis=-1, keepdims=True) + _EPS)


def _artanh(x):
    x = jnp.clip(x, -1.0 + 1e-7, 1.0 - 1e-7)
    return 0.5 * jnp.log((1.0 + x) / (1.0 - x))


def _expmap0(u):
    n = _nrm(u)
    return jnp.tanh(n) * u / n


def _logmap0(p):
    n = _nrm(p)
    return _artanh(n) * p / n


def _proj(x):
    maxnorm = 1.0 - 1e-5
    n = _nrm(x)
    return jnp.where(n > maxnorm, x / n * maxnorm, x)


def _mobius_add(x, y):
    x2 = jnp.sum(x * x, -1, keepdims=True)
    y2 = jnp.sum(y * y, -1, keepdims=True)
    xy = jnp.sum(x * y, -1, keepdims=True)
    num = (1.0 + 2.0 * xy + y2) * x + (1.0 - x2) * y
    den = 1.0 + 2.0 * xy + x2 * y2
    return num / jnp.maximum(den, _EPS)


def _hyplinear_log(x, W, b):
    xn = _nrm(x)
    mx = lax.dot_general(x, W, (((1,), (1,)), ((), ())),
                         preferred_element_type=jnp.float32)
    mxn = _nrm(mx)
    mv = _proj(jnp.tanh(mxn / xn * _artanh(xn)) * mx / mxn)
    hb = _proj(_expmap0(b))
    res = _proj(_mobius_add(mv, hb))
    return _logmap0(res)


def _post_agg(acc, deg):
    s = acc[0] + acc[1]
    d = deg[0, :, 0:1] + deg[1, :, 0:1]
    agg = s / jnp.maximum(d, 1.0)
    out = _proj(_expmap0(agg))
    xt = jax.nn.relu(_logmap0(out))
    return _proj(_expmap0(xt))


def _stage1_body(h_ref, W_ref, b_ref, o_ref):
    x = _proj(_expmap0(h_ref[...]))
    o_ref[...] = _hyplinear_log(x, W_ref[...], b_ref[...])


def _stage2_body(acc_ref, deg_ref, W_ref, b_ref, o_ref):
    x = _post_agg(acc_ref[...], deg_ref[...])
    o_ref[...] = _hyplinear_log(x, W_ref[...], b_ref[...])


def _stage3_body(acc_ref, deg_ref, ca_ref, o_ref, cnt_ref):
    i = pl.program_id(0)
    x = _post_agg(acc_ref[...], deg_ref[...])
    ht = _logmap0(_proj(x))
    ca = ca_ref[...]
    onehot = (ca == lax.broadcasted_iota(jnp.int32, (_BS, _NSEG), 1)
              ).astype(jnp.float32)

    @pl.when(i == 0)
    def _():
        o_ref[...] = jnp.zeros_like(o_ref)
        cnt_ref[...] = jnp.zeros_like(cnt_ref)

    o_ref[...] += lax.dot_general(onehot, ht, (((0,), (0,)), ((), ())),
                                  preferred_element_type=jnp.float32)
    cnt_ref[...] += lax.dot_general(onehot, jnp.ones((_BS, _D), jnp.float32),
                                    (((0,), (0,)), ((), ())),
                                    preferred_element_type=jnp.float32)

    @pl.when(i == pl.num_programs(0) - 1)
    def _():
        mean = o_ref[...] / jnp.maximum(cnt_ref[...], 1.0)
        o_ref[...] = _proj(_expmap0(mean))


_stage1 = pl.pallas_call(
    _stage1_body,
    grid=(_N // _BS,),
    in_specs=[pl.BlockSpec((_BS, _D), lambda i: (i, 0)),
              pl.BlockSpec((_D, _D), lambda i: (0, 0)),
              pl.BlockSpec((1, _D), lambda i: (0, 0))],
    out_specs=pl.BlockSpec((_BS, _D), lambda i: (i, 0)),
    out_shape=jax.ShapeDtypeStruct((_N, _D), jnp.float32),
)

_stage2 = pl.pallas_call(
    _stage2_body,
    grid=(_N // _BS,),
    in_specs=[pl.BlockSpec((2, _BS, _D), lambda i: (0, i, 0)),
              pl.BlockSpec((2, _BS, _D), lambda i: (0, i, 0)),
              pl.BlockSpec((_D, _D), lambda i: (0, 0)),
              pl.BlockSpec((1, _D), lambda i: (0, 0))],
    out_specs=pl.BlockSpec((_BS, _D), lambda i: (i, 0)),
    out_shape=jax.ShapeDtypeStruct((_N, _D), jnp.float32),
)

_stage3 = pl.pallas_call(
    _stage3_body,
    grid=(_N // _BS,),
    in_specs=[pl.BlockSpec((2, _BS, _D), lambda i: (0, i, 0)),
              pl.BlockSpec((2, _BS, _D), lambda i: (0, i, 0)),
              pl.BlockSpec((_BS, 1), lambda i: (i, 0))],
    out_specs=pl.BlockSpec((_NSEG, _D), lambda i: (0, 0)),
    out_shape=jax.ShapeDtypeStruct((_NSEG, _D), jnp.float32),
    scratch_shapes=[pltpu.VMEM((_NSEG, _D), jnp.float32)],
)


_NBUF = 5
_GLEAD = 4
_KD = 80
_NCHD = _EW // _KD
_DWIN = 8


def _sc_agg_body(xt_hbm, src_hbm, dst_hbm, znd_hbm, acc_out,
                 src_v, dst_v, r0, r1, r2, r3, r4, acc_sh,
                 g0, g1, g2, g3, g4, s0, s1, s2, s3, s4):
    cid = lax.axis_index("c")
    sid = lax.axis_index("s")
    row0 = sid * _RPS
    wid = sid * 2 + cid
    rows = (r0, r1, r2, r3, r4)
    gsem = (g0, g1, g2, g3, g4)
    ssem = (s0, s1, s2, s3, s4)

    pltpu.sync_copy(src_hbm.at[pl.ds(wid * _EW, _EW)], src_v)
    pltpu.sync_copy(dst_hbm.at[pl.ds(wid * _EW, _EW)], dst_v)

    def _gather(j, b):
        return pltpu.make_async_copy(
            xt_hbm.at[src_v.at[pl.ds(j * _K, _K)]], rows[b], gsem[b])

    def _scatter(j, b):
        return pltpu.make_async_copy(
            rows[b], acc_sh.at[dst_v.at[pl.ds(j * _K, _K)]], ssem[b])

    def _scatter_start(j, b):
        pltpu.async_copy(rows[b], acc_sh.at[dst_v.at[pl.ds(j * _K, _K)]],
                         ssem[b], add=True)

    for c in range(_GLEAD):
        _gather(c, c).start()
    pltpu.sync_copy(znd_hbm.at[pl.ds(row0, _RPS)], acc_sh.at[pl.ds(row0, _RPS)])
    plsc.subcore_barrier()

    _S = _NBUF - _GLEAD

    for j in range(_S):
        _gather(j, j % _NBUF).wait()
        _scatter_start(j, j % _NBUF)
        _gather(j + _GLEAD, (j + _GLEAD) % _NBUF).start()

    @pl.loop(_S, _NCH - _GLEAD, step=_NBUF)
    def _(ci):
        for u in range(_NBUF):
            b = (_S + u) % _NBUF
            bn = (_S + u + _GLEAD) % _NBUF
            j = ci + u
            _gather(j, b).wait()
            _scatter_start(j, b)
            _scatter(j - _S, bn).wait()
            _gather(j + _GLEAD, bn).start()

    for j in range(_NCH - _GLEAD, _NCH):
        b = j % _NBUF
        bn = (j + _GLEAD) % _NBUF
        _gather(j, b).wait()
        _scatter_start(j, b)
        _scatter(j - _S, bn).wait()
    for j in range(_NCH - _S, _NCH):
        _scatter(j, j % _NBUF).wait()

    plsc.subcore_barrier()
    pltpu.sync_copy(acc_sh.at[pl.ds(row0, _RPS)],
                    acc_out.at[cid, pl.ds(row0, _RPS)])


def _sc_deg_body(dst_hbm, znd_hbm, ones_hbm,
                 deg_out, dst_v, ones_v, deg_sh, sem):
    cid = lax.axis_index("c")
    sid = lax.axis_index("s")
    row0 = sid * _RPS
    wid = sid * 2 + cid
    pltpu.sync_copy(znd_hbm.at[pl.ds(row0, _RPS)], deg_sh.at[pl.ds(row0, _RPS)])
    pltpu.sync_copy(ones_hbm, ones_v)
    pltpu.sync_copy(dst_hbm.at[pl.ds(wid * _EW, _EW)], dst_v)
    plsc.subcore_barrier()

    def _sc(j):
        return pltpu.make_async_copy(
            ones_v, deg_sh.at[dst_v.at[pl.ds(j * _KD, _KD)]], sem)

    def _sc_start(j):
        pltpu.async_copy(ones_v, deg_sh.at[dst_v.at[pl.ds(j * _KD, _KD)]],
                         sem, add=True)

    for j in range(_DWIN):
        _sc_start(j)

    @pl.loop(_DWIN, _NCHD)
    def _(j):
        _sc(j - _DWIN).wait()
        _sc_start(j)

    for j in range(_NCHD - _DWIN, _NCHD):
        _sc(j).wait()

    plsc.subcore_barrier()
    pltpu.sync_copy(deg_sh.at[pl.ds(row0, _RPS)],
                    deg_out.at[cid, pl.ds(row0, _RPS)])


@functools.cache
def _get_sc_kernels():
    mesh = plsc.VectorSubcoreMesh(core_axis_name="c", subcore_axis_name="s")
    agg = pl.kernel(
        _sc_agg_body,
        out_type=jax.ShapeDtypeStruct((2, _NP, _D), jnp.float32),
        mesh=mesh,
        scratch_types=(
            [pltpu.VMEM((_EW,), jnp.int32),
             pltpu.VMEM((_EW,), jnp.int32)]
            + [pltpu.VMEM((_K, _D), jnp.float32) for _ in range(_NBUF)]
            + [pltpu.VMEM_SHARED((_NP, _D), jnp.float32)]
            + [pltpu.SemaphoreType.DMA] * (2 * _NBUF)
        ),
    )
    deg = pl.kernel(
        _sc_deg_body,
        out_type=jax.ShapeDtypeStruct((2, _NP, _D), jnp.float32),
        mesh=mesh,
        scratch_types=[
            pltpu.VMEM((_EW,), jnp.int32),
            pltpu.VMEM((_KD, _D), jnp.float32),
            pltpu.VMEM_SHARED((_NP, _D), jnp.float32),
            pltpu.SemaphoreType.DMA,
        ],
    )
    return agg, deg


def kernel(h, W1, b1, W2, b2, edge_index, comment_assignment):
    src = edge_index[0].astype(jnp.int32)
    dst = edge_index[1].astype(jnp.int32)
    ca = comment_assignment.astype(jnp.int32).reshape(_N, 1)
    zeros_nd = jnp.zeros((_NP, _D), jnp.float32)
    ones_k = jnp.ones((_KD, _D), jnp.float32)

    sc_agg, sc_deg = _get_sc_kernels()
    deg = sc_deg(dst, zeros_nd, ones_k)
    xt1 = _stage1(h, W1, b1.reshape(1, _D))
    acc1 = sc_agg(xt1, src, dst, zeros_nd)
    xt2 = _stage2(acc1, deg, W2, b2.reshape(1, _D))
    acc2 = sc_agg(xt2, src, dst, zeros_nd)
    out = _stage3(acc2, deg, ca)
    return out.reshape(_B, _M, _D)
